# Initial kernel scaffold; baseline (speedup 1.0000x reference)
import jax
import jax.numpy as jnp
from jax.experimental import pallas as pl


def kernel(feats, batch):
    raise NotImplementedError("write your pallas kernel here")


# speedup vs baseline: 1.9178x; 1.9178x over previous
import functools

import jax
import jax.numpy as jnp
from jax import lax
from jax.experimental import pallas as pl
from jax.experimental.pallas import tpu as pltpu
from jax.experimental.pallas import tpu_sc as plsc

N = 320000
D = 128
NUM_SEGMENTS = 10000
L = 16
NW = 32
SPW = 320
S_PAD = NW * SPW
CH = 256
NB = N // L
DG = D // L


def _worker_id():
    return lax.axis_index("c") * 16 + lax.axis_index("s")


def _pool_min_kernel(feats_hbm, batch_hbm, out_hbm, acc, fbuf, bbuf, sbuf):
    wid = _worker_id()
    s0 = wid * SPW

    inf16 = jnp.full((L,), jnp.inf, jnp.float32)

    def init_body(r, _):
        for j in range(DG):
            acc[r, pl.ds(j * L, L)] = inf16
        return 0

    lax.fori_loop(0, SPW, init_body, 0)

    def lower_bound(t):
        def step(_, lohi):
            lo, hi = lohi
            mid = (lo + hi) // 2
            pltpu.sync_copy(batch_hbm.at[pl.ds(mid * L, L)], sbuf)
            cnt = jnp.sum((sbuf[...] < t).astype(jnp.int32))
            found = cnt < L
            lo2 = jnp.where(found, lo, mid + 1)
            hi2 = jnp.where(found, mid, hi)
            return (jnp.minimum(lo2, NB - 1), hi2)

        lo, _ = lax.fori_loop(
            0, 15, step, (jnp.int32(0), jnp.int32(NB - 1)))
        pltpu.sync_copy(batch_hbm.at[pl.ds(lo * L, L)], sbuf)
        cnt = jnp.sum((sbuf[...] < t).astype(jnp.int32))
        return lo * L + cnt

    r0 = lower_bound(s0)
    r1 = lower_bound(s0 + SPW)

    def chunk_body(c, carry):
        pltpu.sync_copy(feats_hbm.at[pl.ds(c * CH, CH), :], fbuf)
        pltpu.sync_copy(batch_hbm.at[pl.ds(c * CH, CH)], bbuf)
        i_lo = jnp.maximum(r0 - c * CH, 0)
        i_hi = jnp.minimum(r1 - c * CH, CH)

        lanes = lax.iota(jnp.int32, L)

        def row_body(i, rc):
            raccs, rprev = rc
            blk = bbuf[pl.ds((i // L) * L, L)]
            seg = jnp.max(jnp.where(lanes == i % L, blk, -1)) - s0
            changed = seg != rprev

            @pl.when(changed)
            def _():
                for j in range(DG):
                    acc[rprev, pl.ds(j * L, L)] = raccs[j]

            new = []
            for j in range(DG):
                row = fbuf[i, pl.ds(j * L, L)]
                new.append(jnp.where(changed, row,
                                     jnp.minimum(raccs[j], row)))
            return (tuple(new), seg)

        return lax.fori_loop(i_lo, i_hi, row_body, carry)

    accs0 = tuple(inf16 for _ in range(DG))
    accs, prev = lax.fori_loop(
        r0 // CH, (r1 + CH - 1) // CH, chunk_body, (accs0, jnp.int32(0)))

    for j in range(DG):
        acc[prev, pl.ds(j * L, L)] = accs[j]

    pltpu.sync_copy(acc, out_hbm.at[pl.ds(s0, SPW), :])


@jax.jit
def kernel(feats, batch):
    mesh = plsc.VectorSubcoreMesh(core_axis_name="c", subcore_axis_name="s",
                                  num_cores=2, num_subcores=16)
    run = functools.partial(
        pl.kernel,
        out_type=jax.ShapeDtypeStruct((S_PAD, D), jnp.float32),
        mesh=mesh,
        compiler_params=pltpu.CompilerParams(needs_layout_passes=False),
        scratch_types=[
            pltpu.VMEM((SPW, D), jnp.float32),
            pltpu.VMEM((CH, D), jnp.float32),
            pltpu.VMEM((CH,), jnp.int32),
            pltpu.VMEM((L,), jnp.int32),
        ],
    )(_pool_min_kernel)
    out = run(feats, batch)
    return out[:NUM_SEGMENTS]


# speedup vs baseline: 2.4875x; 1.2971x over previous
import functools

import jax
import jax.numpy as jnp
from jax import lax
from jax.experimental import pallas as pl
from jax.experimental.pallas import tpu as pltpu
from jax.experimental.pallas import tpu_sc as plsc

N = 320000
D = 128
NUM_SEGMENTS = 10000
L = 16
NW = 32
SPW = 320
S_PAD = NW * SPW
CH = 256
NB = N // L
DG = D // L


def _worker_id():
    return lax.axis_index("c") * 16 + lax.axis_index("s")


def _pool_min_kernel(feats_hbm, batch_hbm, out_hbm, acc,
                     fbuf0, fbuf1, bbuf0, bbuf1, sbuf, sem0, sem1):
    wid = _worker_id()
    s0 = wid * SPW

    inf16 = jnp.full((L,), jnp.inf, jnp.float32)

    def init_body(r, _):
        for j in range(DG):
            acc[r, pl.ds(j * L, L)] = inf16
        return 0

    lax.fori_loop(0, SPW, init_body, 0)

    def lower_bound(t):
        def step(_, lohi):
            lo, hi = lohi
            mid = (lo + hi) // 2
            pltpu.sync_copy(batch_hbm.at[pl.ds(mid * L, L)], sbuf)
            cnt = jnp.sum((sbuf[...] < t).astype(jnp.int32))
            found = cnt < L
            lo2 = jnp.where(found, lo, mid + 1)
            hi2 = jnp.where(found, mid, hi)
            return (jnp.minimum(lo2, NB - 1), hi2)

        lo, _ = lax.fori_loop(
            0, 15, step, (jnp.int32(0), jnp.int32(NB - 1)))
        pltpu.sync_copy(batch_hbm.at[pl.ds(lo * L, L)], sbuf)
        cnt = jnp.sum((sbuf[...] < t).astype(jnp.int32))
        return lo * L + cnt

    r0 = lower_bound(s0)
    r1 = lower_bound(s0 + SPW)
    c_lo = r0 // CH
    c_hi = (r1 + CH - 1) // CH

    fbufs = (fbuf0, fbuf1)
    bbufs = (bbuf0, bbuf1)
    sems = (sem0, sem1)

    def start(c, b):
        pltpu.async_copy(feats_hbm.at[pl.ds(c * CH, CH), :], fbufs[b], sems[b])
        pltpu.async_copy(batch_hbm.at[pl.ds(c * CH, CH)], bbufs[b], sems[b])

    def wait(c, b):
        pltpu.make_async_copy(
            feats_hbm.at[pl.ds(c * CH, CH), :], fbufs[b], sems[b]).wait()
        pltpu.make_async_copy(
            batch_hbm.at[pl.ds(c * CH, CH)], bbufs[b], sems[b]).wait()

    lanes = lax.iota(jnp.int32, L)

    def process(c, b, accs_prev):
        i_lo = jnp.maximum(r0 - c * CH, 0)
        i_hi = jnp.minimum(r1 - c * CH, CH)
        fb = fbufs[b]
        bb = bbufs[b]

        def row_body(i, rc):
            raccs, rprev = rc
            blk = bb[pl.ds((i // L) * L, L)]
            seg = jnp.max(jnp.where(lanes == i % L, blk, -1)) - s0
            changed = seg != rprev

            @pl.when(changed)
            def _():
                for j in range(DG):
                    acc[rprev, pl.ds(j * L, L)] = raccs[j]

            new = []
            for j in range(DG):
                row = fb[i, pl.ds(j * L, L)]
                new.append(jnp.where(changed, row,
                                     jnp.minimum(raccs[j], row)))
            return (tuple(new), seg)

        return lax.fori_loop(i_lo, i_hi, row_body, accs_prev)

    @pl.when(c_lo < c_hi)
    def _():
        start(c_lo, 0)

    def pair_body(p, accs_prev):
        for b in (0, 1):
            c = c_lo + 2 * p + b

            @pl.when(c + 1 < c_hi)
            def _():
                start(c + 1, 1 - b)

            def do(ap, c=c, b=b):
                wait(c, b)
                return process(c, b, ap)

            accs_prev = lax.cond(c < c_hi, do, lambda ap: ap, accs_prev)
        return accs_prev

    accs0 = tuple(inf16 for _ in range(DG))
    npairs = (c_hi - c_lo + 1) // 2
    accs, prev = lax.fori_loop(0, npairs, pair_body, (accs0, jnp.int32(0)))

    for j in range(DG):
        acc[prev, pl.ds(j * L, L)] = accs[j]

    pltpu.sync_copy(acc, out_hbm.at[pl.ds(s0, SPW), :])


def _build(mesh=None, interpret=False):
    if mesh is None:
        mesh = plsc.VectorSubcoreMesh(core_axis_name="c",
                                      subcore_axis_name="s",
                                      num_cores=2, num_subcores=16)
    return functools.partial(
        pl.kernel,
        out_type=jax.ShapeDtypeStruct((S_PAD, D), jnp.float32),
        mesh=mesh,
        compiler_params=pltpu.CompilerParams(needs_layout_passes=False),
        scratch_types=[
            pltpu.VMEM((SPW, D), jnp.float32),
            pltpu.VMEM((CH, D), jnp.float32),
            pltpu.VMEM((CH, D), jnp.float32),
            pltpu.VMEM((CH,), jnp.int32),
            pltpu.VMEM((CH,), jnp.int32),
            pltpu.VMEM((L,), jnp.int32),
            pltpu.SemaphoreType.DMA,
            pltpu.SemaphoreType.DMA,
        ],
        interpret=interpret,
    )(_pool_min_kernel)


@jax.jit
def kernel(feats, batch):
    out = _build()(feats, batch)
    return out[:NUM_SEGMENTS]
